# baseline (speedup 1.0000x reference)
import dataclasses
import functools

import jax
import jax.numpy as jnp
from jax import lax
from jax.experimental import pallas as pl
from jax.experimental.pallas import tpu as pltpu
from jax.experimental.pallas import tpu_sc as plsc

NN = 50000
EE = 800000
NC, NS, LN = 2, 16, 16
NW = NC * NS
NP = 50048
NPR = NP // 128
ROWS = NP // NS
EW = EE // NW
WIN = 1000
WINR = 200
ER = EE // 128
F32 = jnp.float32


def _mesh():
    return plsc.VectorSubcoreMesh(core_axis_name="c", subcore_axis_name="s",
                                  num_cores=NC, num_subcores=NS)


def _sc_params():
    cp = pltpu.CompilerParams()
    fields = pltpu.CompilerParams.__dataclass_fields__
    if "needs_layout_passes" in fields:
        cp = dataclasses.replace(cp, needs_layout_passes=False)
    if "use_tc_tiling_on_sc" in fields:
        cp = dataclasses.replace(cp, use_tc_tiling_on_sc=False)
    return cp


def _sc_gather_scalars(pairs):
    P = len(pairs)
    tables = [t for t, _ in pairs]
    idxs = [i for _, i in pairs]

    @functools.partial(
        pl.kernel,
        out_type=[jax.ShapeDtypeStruct((EE,), F32)] * P,
        mesh=_mesh(),
        compiler_params=_sc_params(),
        scratch_types=[pltpu.VMEM((NP,), F32),
                       pltpu.VMEM((WIN,), jnp.int32),
                       pltpu.VMEM((WIN,), F32)],
    )
    def k(*refs):
        tabs = refs[:P]
        idx_h = refs[P:2 * P]
        outs = refs[2 * P:3 * P]
        tab_v, idx_v, val_v = refs[3 * P:]
        wid = lax.axis_index("s") * NC + lax.axis_index("c")
        base = wid * EW
        for p in range(P):
            pltpu.sync_copy(tabs[p], tab_v)

            @pl.loop(0, EW, step=WIN)
            def _(off, p=p):
                pltpu.sync_copy(idx_h[p].at[pl.ds(base + off, WIN)], idx_v)

                @pl.loop(0, WIN - LN, step=LN)
                def _(j):
                    val_v[pl.ds(j, LN)] = plsc.load_gather(
                        tab_v, [idx_v[pl.ds(j, LN)]])

                val_v[pl.ds(WIN - LN, LN)] = plsc.load_gather(
                    tab_v, [idx_v[pl.ds(WIN - LN, LN)]])

                pltpu.sync_copy(val_v, outs[p].at[pl.ds(base + off, WIN)])

    res = k(*tables, *idxs)
    return res if P > 1 else [res]


def _sc_scatter_scalars(vals_list, idx, zeros_np):
    P = len(vals_list)

    @functools.partial(
        pl.kernel,
        out_type=[jax.ShapeDtypeStruct((NC * NP,), F32)] * P,
        mesh=_mesh(),
        compiler_params=_sc_params(),
        scratch_types=[pltpu.VMEM_SHARED((NP,), F32)] * P
        + [pltpu.VMEM((WIN,), jnp.int32), pltpu.VMEM((WIN,), F32),
           pltpu.VMEM((ROWS,), F32)],
    )
    def k(*refs):
        z_h = refs[0]
        vals_h = refs[1:1 + P]
        idx_h = refs[1 + P]
        outs = refs[2 + P:2 + 2 * P]
        accs = refs[2 + 2 * P:2 + 3 * P]
        idx_v = refs[2 + 3 * P]
        val_v = refs[3 + 3 * P]
        row_v = refs[4 + 3 * P]
        cid = lax.axis_index("c")
        sid = lax.axis_index("s")
        pltpu.sync_copy(z_h.at[pl.ds(sid * ROWS, ROWS)], row_v)
        for p in range(P):
            pltpu.sync_copy(row_v, accs[p].at[pl.ds(sid * ROWS, ROWS)])
        plsc.subcore_barrier()
        base = (sid * NC + cid) * EW

        @pl.loop(0, EW, step=WIN)
        def _(off):
            pltpu.sync_copy(idx_h.at[pl.ds(base + off, WIN)], idx_v)
            for p in range(P):
                pltpu.sync_copy(vals_h[p].at[pl.ds(base + off, WIN)], val_v)
                pltpu.sync_copy(val_v, accs[p].at[idx_v], add=True)

        plsc.subcore_barrier()
        for p in range(P):
            pltpu.sync_copy(accs[p].at[pl.ds(sid * ROWS, ROWS)], row_v)
            pltpu.sync_copy(row_v,
                            outs[p].at[pl.ds(cid * NP + sid * ROWS, ROWS)])

    res = k(zeros_np, *vals_list, idx)
    return res if P > 1 else [res]


def _sc_edge_prolog(xp, src, dst, ea, ones_e, zeros_np):
    @functools.partial(
        pl.kernel,
        out_type=[jax.ShapeDtypeStruct((EE,), F32)] * 2
        + [jax.ShapeDtypeStruct((NC * NP,), F32)] * 2,
        mesh=_mesh(),
        compiler_params=_sc_params(),
        scratch_types=[pltpu.VMEM_SHARED((NP,), F32),
                       pltpu.VMEM_SHARED((NP,), F32),
                       pltpu.VMEM((NP,), F32),
                       pltpu.VMEM((WIN,), jnp.int32),
                       pltpu.VMEM((WIN,), F32),
                       pltpu.VMEM((ROWS,), F32)],
    )
    def k(x_h, src_h, dst_h, ea_h, one_h, z_h,
          xs_h, xd_h, degP_h, asumP_h,
          acc0, acc1, tab_v, idx_v, val_v, row_v):
        cid = lax.axis_index("c")
        sid = lax.axis_index("s")
        pltpu.sync_copy(z_h.at[pl.ds(sid * ROWS, ROWS)], row_v)
        pltpu.sync_copy(row_v, acc0.at[pl.ds(sid * ROWS, ROWS)])
        pltpu.sync_copy(row_v, acc1.at[pl.ds(sid * ROWS, ROWS)])
        pltpu.sync_copy(x_h, tab_v)
        plsc.subcore_barrier()
        base = (sid * NC + cid) * EW

        @pl.loop(0, EW, step=WIN)
        def _(off):
            for (ih, oh) in ((src_h, xs_h), (dst_h, xd_h)):
                pltpu.sync_copy(ih.at[pl.ds(base + off, WIN)], idx_v)

                @pl.loop(0, WIN - LN, step=LN)
                def _(j):
                    val_v[pl.ds(j, LN)] = plsc.load_gather(
                        tab_v, [idx_v[pl.ds(j, LN)]])

                val_v[pl.ds(WIN - LN, LN)] = plsc.load_gather(
                    tab_v, [idx_v[pl.ds(WIN - LN, LN)]])
                pltpu.sync_copy(val_v, oh.at[pl.ds(base + off, WIN)])

            pltpu.sync_copy(one_h.at[pl.ds(base + off, WIN)], val_v)
            pltpu.sync_copy(val_v, acc0.at[idx_v], add=True)
            pltpu.sync_copy(ea_h.at[pl.ds(base + off, WIN)], val_v)
            pltpu.sync_copy(val_v, acc1.at[idx_v], add=True)

        plsc.subcore_barrier()
        for acc, out in ((acc0, degP_h), (acc1, asumP_h)):
            pltpu.sync_copy(acc.at[pl.ds(sid * ROWS, ROWS)], row_v)
            pltpu.sync_copy(row_v,
                            out.at[pl.ds(cid * NP + sid * ROWS, ROWS)])

    return k(xp, src, dst, ea, ones_e, zeros_np)


def _sc_gather_rows(pairs):
    P = len(pairs)
    tables = [t for t, _ in pairs]
    idxs = [i for _, i in pairs]

    @functools.partial(
        pl.kernel,
        out_type=[jax.ShapeDtypeStruct((EE, 32), F32)] * P,
        mesh=_mesh(),
        compiler_params=_sc_params(),
        scratch_types=[pltpu.VMEM_SHARED((NP, 32), F32),
                       pltpu.VMEM((WINR,), jnp.int32),
                       pltpu.VMEM((WINR, 32), F32),
                       pltpu.VMEM((184, 32), F32)],
    )
    def k(*refs):
        tabs = refs[:P]
        idx_h = refs[P:2 * P]
        outs = refs[2 * P:3 * P]
        tab_sh, idx_v, row_v, stage_v = refs[3 * P:]
        cid = lax.axis_index("c")
        sid = lax.axis_index("s")
        base = (sid * NC + cid) * EW
        ch = 184
        for p in range(P):
            @pl.loop(0, ROWS, step=ch)
            def _(r, p=p):
                pltpu.sync_copy(tabs[p].at[pl.ds(sid * ROWS + r, ch)],
                                stage_v)
                pltpu.sync_copy(stage_v,
                                tab_sh.at[pl.ds(sid * ROWS + r, ch)])

            plsc.subcore_barrier()

            @pl.loop(0, EW, step=WINR)
            def _(off, p=p):
                pltpu.sync_copy(idx_h[p].at[pl.ds(base + off, WINR)], idx_v)
                pltpu.sync_copy(tab_sh.at[idx_v], row_v)
                pltpu.sync_copy(row_v, outs[p].at[pl.ds(base + off, WINR)])

            plsc.subcore_barrier()

    return k(*tables, *idxs)


def _sc_scatter_rows(wu0, wu1, ex0, ex1, idx, zeros_rows, zeros_np):
    @functools.partial(
        pl.kernel,
        out_type=[jax.ShapeDtypeStruct((NP, 32), F32)] * 2
        + [jax.ShapeDtypeStruct((NP,), F32)] * 2,
        mesh=_mesh(),
        compiler_params=_sc_params(),
        scratch_types=[pltpu.VMEM_SHARED((NP, 32), F32),
                       pltpu.VMEM_SHARED((NP,), F32),
                       pltpu.VMEM((WINR,), jnp.int32),
                       pltpu.VMEM((WINR, 32), F32),
                       pltpu.VMEM((WINR,), F32),
                       pltpu.VMEM((184, 32), F32),
                       pltpu.VMEM((ROWS,), F32)],
    )
    def k(z_h, zn_h, wu0_h, wu1_h, ex0_h, ex1_h, idx_h,
          out0_h, out1_h, den0_h, den1_h,
          acc, dacc, idx_v, row_v, val_v, stage_v, nstage_v):
        cid = lax.axis_index("c")
        sid = lax.axis_index("s")
        ch = 184

        @pl.loop(0, ROWS, step=ch)
        def _(r):
            pltpu.sync_copy(z_h.at[pl.ds(sid * ROWS + r, ch)], stage_v)
            pltpu.sync_copy(stage_v, acc.at[pl.ds(sid * ROWS + r, ch)])

        pltpu.sync_copy(zn_h.at[pl.ds(sid * ROWS, ROWS)], nstage_v)
        pltpu.sync_copy(nstage_v, dacc.at[pl.ds(sid * ROWS, ROWS)])
        plsc.subcore_barrier()
        ew = EE // NS
        base = sid * ew

        def run(wu_h, ex_h, out_h, den_h):
            @pl.loop(0, ew, step=WINR)
            def _(off):
                pltpu.sync_copy(idx_h.at[pl.ds(base + off, WINR)], idx_v)
                pltpu.sync_copy(wu_h.at[pl.ds(base + off, WINR)], row_v)
                pltpu.sync_copy(row_v, acc.at[idx_v], add=True)
                pltpu.sync_copy(ex_h.at[pl.ds(base + off, WINR)], val_v)
                pltpu.sync_copy(val_v, dacc.at[idx_v], add=True)

            plsc.subcore_barrier()

            @pl.loop(0, ROWS, step=ch)
            def _(r):
                pltpu.sync_copy(acc.at[pl.ds(sid * ROWS + r, ch)], stage_v)
                pltpu.sync_copy(stage_v, out_h.at[pl.ds(sid * ROWS + r, ch)])

            pltpu.sync_copy(dacc.at[pl.ds(sid * ROWS, ROWS)], nstage_v)
            pltpu.sync_copy(nstage_v, den_h.at[pl.ds(sid * ROWS, ROWS)])

        @pl.when(cid == 0)
        def _():
            run(wu0_h, ex0_h, out0_h, den0_h)

        @pl.when(cid == 1)
        def _():
            run(wu1_h, ex1_h, out1_h, den1_h)

    return k(zeros_rows, zeros_np, wu0, wu1, ex0, ex1, idx)


def _leaky(m):
    return jnp.where(m >= 0, m, 0.2 * m)


def _tc_elementwise(body, ins, out_shapes, specs_in, specs_out, grid):
    return pl.pallas_call(
        body,
        grid=grid,
        in_specs=specs_in,
        out_specs=specs_out,
        out_shape=[jax.ShapeDtypeStruct(s, F32) for s in out_shapes],
    )(*ins)


def _node_spec():
    return pl.BlockSpec((NPR, 128), lambda: (0, 0))


def _wspec():
    return pl.BlockSpec((1, 64), lambda: (0, 0))


def tc_node1(degP0, degP1, asumP0, asumP1, xp, wlr, we, bb, att):
    def body(d0, d1, a0, a1, x, wlr_r, we_r, bb_r, att_r,
             cnt_o, la_o, n0_o, n1_o):
        deg = d0[...] + d1[...]
        asum = a0[...] + a1[...]
        la = asum / jnp.maximum(deg, 1.0)
        cnt_o[...] = deg + 1.0
        la_o[...] = la
        x_ = x[...]
        acc0 = jnp.zeros_like(x_)
        acc1 = jnp.zeros_like(x_)
        for kk in range(64):
            m = x_ * wlr_r[0, kk] + la * we_r[0, kk] + bb_r[0, kk]
            m = _leaky(m) * att_r[0, kk]
            if kk < 32:
                acc0 = acc0 + m
            else:
                acc1 = acc1 + m
        n0_o[...] = acc0
        n1_o[...] = acc1

    return _tc_elementwise(
        body,
        [degP0, degP1, asumP0, asumP1, xp, wlr, we, bb, att],
        [(NPR, 128)] * 4,
        [_node_spec()] * 5 + [_wspec()] * 4,
        [_node_spec()] * 4,
        grid=(),
    )


_EB = 8192
_EBR = _EB // 128
_EGRID = (EE + _EB - 1) // _EB


def _edge_spec():
    return pl.BlockSpec((_EBR, 128), lambda i: (i, 0))


def tc_alpha1(xs, xd, ea, wl, wr, we, bb, att):
    def body(xs_r, xd_r, ea_r, wl_r, wr_r, we_r, bb_r, att_r, a0_o, a1_o):
        xs_ = xs_r[...]
        xd_ = xd_r[...]
        ea_ = ea_r[...]
        acc0 = jnp.zeros_like(xs_)
        acc1 = jnp.zeros_like(xs_)
        for kk in range(64):
            m = (xs_ * wl_r[0, kk] + xd_ * wr_r[0, kk]
                 + ea_ * we_r[0, kk] + bb_r[0, kk])
            m = _leaky(m) * att_r[0, kk]
            if kk < 32:
                acc0 = acc0 + m
            else:
                acc1 = acc1 + m
        a0_o[...] = acc0
        a1_o[...] = acc1

    return _tc_elementwise(
        body,
        [xs, xd, ea, wl, wr, we, bb, att],
        [(ER, 128)] * 2,
        [_edge_spec()] * 3 + [pl.BlockSpec((1, 64), lambda i: (0, 0))] * 5,
        [_edge_spec()] * 2,
        grid=(_EGRID,),
    )


def tc_mean1(salP0_0, salP1_0, salP0_1, salP1_1, n0, n1, cnt, xp):
    def body(s00, s10, s01, s11, n0_r, n1_r, cnt_r, x_r,
             m0_o, m1_o, id0_o, is0_o, id1_o, is1_o):
        cnt_ = cnt_r[...]
        x_ = x_r[...]
        for (sa, sb, n_r, m_o, id_o, is_o) in (
                (s00, s10, n0_r, m0_o, id0_o, is0_o),
                (s01, s11, n1_r, m1_o, id1_o, is1_o)):
            n_ = n_r[...]
            mean = (sa[...] + sb[...] + n_) / cnt_
            exn = jnp.exp(n_ - mean)
            m_o[...] = mean
            id_o[...] = exn
            is_o[...] = exn * x_

    return _tc_elementwise(
        body,
        [salP0_0, salP1_0, salP0_1, salP1_1, n0, n1, cnt, xp],
        [(NPR, 128)] * 6,
        [_node_spec()] * 8,
        [_node_spec()] * 6,
        grid=(),
    )


def tc_exp1(a0, a1, md0, md1, xs):
    def body(a0_r, a1_r, m0_r, m1_r, xs_r, e0_o, es0_o, e1_o, es1_o):
        xs_ = xs_r[...]
        e0 = jnp.exp(a0_r[...] - m0_r[...])
        e1 = jnp.exp(a1_r[...] - m1_r[...])
        e0_o[...] = e0
        es0_o[...] = e0 * xs_
        e1_o[...] = e1
        es1_o[...] = e1 * xs_

    return _tc_elementwise(
        body,
        [a0, a1, md0, md1, xs],
        [(ER, 128)] * 4,
        [_edge_spec()] * 5,
        [_edge_spec()] * 4,
        grid=(_EGRID,),
    )


_BN = 6144
_BNR = _BN // 128
_NGRID = (NP + _BN - 1) // _BN


def _colspec():
    return pl.BlockSpec((1, _BN), lambda i: (0, i))


def _rowspec32():
    return pl.BlockSpec((_BN, 32), lambda i: (i, 0))


def _col(v):
    return jnp.transpose(v[...])


def tc_h1(denP00, denP10, SP00, SP10, denP01, denP11, SP01, SP11,
          iden0, iS0, iden1, iS1, la,
          Wl1, bl1, bias1, Wl2, bl2, Wr2, br2, We2, att2):
    def body(d00, d10, s00, s10, d01, d11, s01, s11,
             i0, j0, i1, j1, la_r,
             wl1_r, bl1_r, b1_r, wl2_r, bl2_r, wr2_r, br2_r, we2_r, att2_r,
             u0_o, u1_o, v0_o, v1_o, a2n0_o, a2n1_o):
        den0 = _col(d00) + _col(d10) + _col(i0)
        S0 = _col(s00) + _col(s10) + _col(j0)
        den1 = _col(d01) + _col(d11) + _col(i1)
        S1 = _col(s01) + _col(s11) + _col(j1)
        A0 = S0 / (den0 + 1e-16)
        SA0 = den0 / (den0 + 1e-16)
        A1 = S1 / (den1 + 1e-16)
        SA1 = den1 / (den1 + 1e-16)
        wl1 = wl1_r[...]
        bl1_ = bl1_r[...]
        h1a = A0 * wl1[:, :32] + SA0 * bl1_[:, :32] + b1_r[:, :32]
        h1b = A1 * wl1[:, 32:] + SA1 * bl1_[:, 32:] + b1_r[:, 32:]
        h1 = jnp.maximum(jnp.concatenate([h1a, h1b], axis=1), 0.0)
        xl2 = jax.lax.dot_general(
            h1, wl2_r[...], (((1,), (0,)), ((), ())),
            preferred_element_type=F32) + bl2_r[...]
        xr2 = jax.lax.dot_general(
            h1, wr2_r[...], (((1,), (0,)), ((), ())),
            preferred_element_type=F32) + br2_r[...]
        u0_o[...] = xl2[:, :32]
        u1_o[...] = xl2[:, 32:]
        v0_o[...] = xr2[:, :32]
        v1_o[...] = xr2[:, 32:]
        m2n = _leaky(xl2 + xr2 + _col(la_r) * we2_r[...]) * att2_r[...]
        a2n0 = jnp.sum(m2n[:, :32], axis=1, keepdims=True)
        a2n1 = jnp.sum(m2n[:, 32:], axis=1, keepdims=True)
        a2n0_o[...] = jnp.transpose(a2n0)
        a2n1_o[...] = jnp.transpose(a2n1)

    wspec = pl.BlockSpec((1, 64), lambda i: (0, 0))
    mspec = pl.BlockSpec((64, 64), lambda i: (0, 0))
    return pl.pallas_call(
        body,
        grid=(_NGRID,),
        in_specs=[_colspec()] * 13 + [wspec, wspec, wspec, mspec, wspec,
                                      mspec, wspec, wspec, wspec],
        out_specs=[_rowspec32()] * 4 + [_colspec()] * 2,
        out_shape=[jax.ShapeDtypeStruct((NP, 32), F32)] * 4
        + [jax.ShapeDtypeStruct((1, NP), F32)] * 2,
    )(denP00, denP10, SP00, SP10, denP01, denP11, SP01, SP11,
      iden0, iS0, iden1, iS1, la,
      Wl1, bl1, bias1, Wl2, bl2, Wr2, br2, We2, att2)


def _erowspec():
    return pl.BlockSpec((_EB, 32), lambda i: (i, 0))


def _escalarspec():
    return pl.BlockSpec((1, _EB), lambda i: (0, i))


def tc_alpha2(u0, v0, u1, v1, ea, We2, att2):
    def body(u0_r, v0_r, u1_r, v1_r, ea_r, we_r, att_r, a0_o, a1_o):
        eac = jnp.transpose(ea_r[...])
        we = we_r[...]
        att = att_r[...]
        m0 = _leaky(u0_r[...] + v0_r[...] + eac * we[:, :32]) * att[:, :32]
        m1 = _leaky(u1_r[...] + v1_r[...] + eac * we[:, 32:]) * att[:, 32:]
        a0 = jnp.sum(m0, axis=1, keepdims=True)
        a1 = jnp.sum(m1, axis=1, keepdims=True)
        a0_o[...] = jnp.transpose(a0)
        a1_o[...] = jnp.transpose(a1)

    wspec = pl.BlockSpec((1, 64), lambda i: (0, 0))
    return pl.pallas_call(
        body,
        grid=(_EGRID,),
        in_specs=[_erowspec()] * 4 + [_escalarspec(), wspec, wspec],
        out_specs=[_escalarspec()] * 2,
        out_shape=[jax.ShapeDtypeStruct((1, EE), F32)] * 2,
    )(u0, v0, u1, v1, ea, We2, att2)


def tc_mean2(salP0_0, salP1_0, salP0_1, salP1_1, n0, n1, cnt):
    def body(s00, s10, s01, s11, n0_r, n1_r, cnt_r,
             m0_o, m1_o, x0_o, x1_o):
        cnt_ = cnt_r[...]
        for (sa, sb, n_r, m_o, x_o) in ((s00, s10, n0_r, m0_o, x0_o),
                                        (s01, s11, n1_r, m1_o, x1_o)):
            n_ = n_r[...]
            mean = (sa[...] + sb[...] + n_) / cnt_
            m_o[...] = mean
            x_o[...] = jnp.exp(n_ - mean)

    return _tc_elementwise(
        body,
        [salP0_0, salP1_0, salP0_1, salP1_1, n0, n1, cnt],
        [(NPR, 128)] * 4,
        [_node_spec()] * 7,
        [_node_spec()] * 4,
        grid=(),
    )


def tc_w2(a0, a1, md0, md1, u0, u1):
    def body(a0_r, a1_r, m0_r, m1_r, u0_r, u1_r,
             e0_o, e1_o, w0_o, w1_o):
        e0 = jnp.exp(a0_r[...] - m0_r[...])
        e1 = jnp.exp(a1_r[...] - m1_r[...])
        e0_o[...] = e0
        e1_o[...] = e1
        w0_o[...] = u0_r[...] * jnp.transpose(e0)
        w1_o[...] = u1_r[...] * jnp.transpose(e1)

    return pl.pallas_call(
        body,
        grid=(_EGRID,),
        in_specs=[_escalarspec()] * 4 + [_erowspec()] * 2,
        out_specs=[_escalarspec()] * 2 + [_erowspec()] * 2,
        out_shape=[jax.ShapeDtypeStruct((1, EE), F32)] * 2
        + [jax.ShapeDtypeStruct((EE, 32), F32)] * 2,
    )(a0, a1, md0, md1, u0, u1)


def tc_readout(r0, r1, d2_0, d2_1, xn0, xn1, u0, u1,
               bias2, message, Wfc, bfc):
    def body(r0_r, r1_r, d00, d01, x0, x1, u0_r, u1_r,
             b2_r, msg_r, wfc_r, bfc_r, out_o):
        xn0_ = _col(x0)
        xn1_ = _col(x1)
        den0 = _col(d00) + xn0_ + 1e-16
        den1 = _col(d01) + xn1_ + 1e-16
        o0 = (r0_r[...] + xn0_ * u0_r[...]) / den0
        o1 = (r1_r[...] + xn1_ * u1_r[...]) / den1
        h2 = jnp.maximum(
            jnp.concatenate([o0, o1], axis=1) + b2_r[...], 0.0)
        me = jax.lax.dot_general(
            msg_r[...], wfc_r[...], (((1,), (1,)), ((), ())),
            preferred_element_type=F32) + bfc_r[...]
        dots = jax.lax.dot_general(
            h2, me, (((1,), (1,)), ((), ())), preferred_element_type=F32)
        mx = jnp.max(dots, axis=1, keepdims=True)
        p = jnp.exp(dots - mx)
        out_o[...] = p / jnp.sum(p, axis=1, keepdims=True)

    wspec = pl.BlockSpec((1, 64), lambda i: (0, 0))
    return pl.pallas_call(
        body,
        grid=(_NGRID,),
        in_specs=[_rowspec32()] * 2 + [_colspec()] * 4 + [_rowspec32()] * 2
        + [wspec,
           pl.BlockSpec((256, 128), lambda i: (0, 0)),
           pl.BlockSpec((64, 128), lambda i: (0, 0)),
           wspec],
        out_specs=pl.BlockSpec((_BN, 256), lambda i: (i, 0)),
        out_shape=jax.ShapeDtypeStruct((NN, 256), F32),
    )(r0, r1, d2_0, d2_1, xn0, xn1, u0, u1,
      bias2, message, Wfc, bfc)


def kernel(message, x, edge_index, edge_attr, Wl1, bl1, Wr1, br1, We1, att1,
           bias1, Wl2, bl2, Wr2, br2, We2, att2, bias2, Wfc, bfc):
    xp = jnp.pad(x[:, 0], (0, NP - NN))
    src = edge_index[0]
    dst = edge_index[1]
    ea = edge_attr[:, 0]
    zeros_np = jnp.zeros((NP,), F32)
    zeros_rows = jnp.zeros((NP, 32), F32)
    ones_e = jnp.ones((EE,), F32)

    nv = lambda a: a.reshape(NPR, 128)
    rv = lambda a: a.reshape(1, NP)
    ev = lambda a: a.reshape(ER, 128)
    er = lambda a: a.reshape(1, EE)
    w = lambda a: a.reshape(1, 64)

    wl1 = w(Wl1)
    wr1 = w(Wr1)
    we1 = w(We1)
    bb1 = w(bl1 + br1)
    at1 = w(att1.reshape(-1))
    we2 = w(We2)
    at2 = w(att2.reshape(-1))

    xs, xd, degP, asumP = _sc_edge_prolog(xp, src, dst, ea, ones_e, zeros_np)
    cnt, la, n1_0, n1_1 = tc_node1(
        nv(degP[:NP]), nv(degP[NP:]), nv(asumP[:NP]), nv(asumP[NP:]),
        nv(xp), w(Wl1 + Wr1), we1, bb1, at1)
    a1_0, a1_1 = tc_alpha1(ev(xs), ev(xd), ev(ea), wl1, wr1, we1, bb1, at1)
    salP_0, salP_1 = _sc_scatter_scalars(
        [a1_0.reshape(-1), a1_1.reshape(-1)], dst, zeros_np)
    m1_0, m1_1, iden0, iS0, iden1, iS1 = tc_mean1(
        nv(salP_0[:NP]), nv(salP_0[NP:]), nv(salP_1[:NP]), nv(salP_1[NP:]),
        n1_0, n1_1, cnt, nv(xp))
    md0, md1 = _sc_gather_scalars(
        [(m1_0.reshape(-1), dst), (m1_1.reshape(-1), dst)])
    ex0, exs0, ex1, exs1 = tc_exp1(a1_0, a1_1, ev(md0), ev(md1), ev(xs))
    denP_0, SP_0, denP_1, SP_1 = _sc_scatter_scalars(
        [ex0.reshape(-1), exs0.reshape(-1), ex1.reshape(-1),
         exs1.reshape(-1)], dst, zeros_np)

    u0t, u1t, v0t, v1t, a2n0, a2n1 = tc_h1(
        rv(denP_0[:NP]), rv(denP_0[NP:]), rv(SP_0[:NP]), rv(SP_0[NP:]),
        rv(denP_1[:NP]), rv(denP_1[NP:]), rv(SP_1[:NP]), rv(SP_1[NP:]),
        rv(iden0), rv(iS0), rv(iden1), rv(iS1), rv(la),
        wl1, w(bl1), w(bias1), Wl2, w(bl2), Wr2, w(br2), we2, at2)

    u0g, u1g, v0g, v1g = _sc_gather_rows(
        [(u0t, src), (u1t, src), (v0t, dst), (v1t, dst)])
    a2_0, a2_1 = tc_alpha2(u0g, v0g, u1g, v1g, er(ea), we2, at2)
    sal2_0, sal2_1 = _sc_scatter_scalars(
        [a2_0.reshape(-1), a2_1.reshape(-1)], dst, zeros_np)
    m2_0, m2_1, xn2_0, xn2_1 = tc_mean2(
        nv(sal2_0[:NP]), nv(sal2_0[NP:]), nv(sal2_1[:NP]), nv(sal2_1[NP:]),
        nv(a2n0), nv(a2n1), cnt)
    md2_0, md2_1 = _sc_gather_scalars(
        [(m2_0.reshape(-1), dst), (m2_1.reshape(-1), dst)])
    ex2_0, ex2_1, wu0, wu1 = tc_w2(a2_0, a2_1, er(md2_0), er(md2_1),
                                   u0g, u1g)
    r0, r1, den2_0, den2_1 = _sc_scatter_rows(
        wu0, wu1, ex2_0.reshape(-1), ex2_1.reshape(-1), dst,
        zeros_rows, zeros_np)

    return tc_readout(
        r0, r1, rv(den2_0), rv(den2_1),
        rv(xn2_0), rv(xn2_1), u0t, u1t,
        w(bias2), message, Wfc, w(bfc))


# speedup vs baseline: 1.0401x; 1.0401x over previous
import dataclasses
import functools

import jax
import jax.numpy as jnp
from jax import lax
from jax.experimental import pallas as pl
from jax.experimental.pallas import tpu as pltpu
from jax.experimental.pallas import tpu_sc as plsc

NN = 50000
EE = 800000
NC, NS, LN = 2, 16, 16
NW = NC * NS
NP = 50048
NPR = NP // 128
ROWS = NP // NS
EW = EE // NW
WIN = 1000
WINR = 200
ER = EE // 128
F32 = jnp.float32


def _mesh():
    return plsc.VectorSubcoreMesh(core_axis_name="c", subcore_axis_name="s",
                                  num_cores=NC, num_subcores=NS)


def _sc_params():
    cp = pltpu.CompilerParams()
    fields = pltpu.CompilerParams.__dataclass_fields__
    if "needs_layout_passes" in fields:
        cp = dataclasses.replace(cp, needs_layout_passes=False)
    if "use_tc_tiling_on_sc" in fields:
        cp = dataclasses.replace(cp, use_tc_tiling_on_sc=False)
    return cp


def _sc_gather_scalars(pairs):
    P = len(pairs)
    tables = [t for t, _ in pairs]
    idxs = [i for _, i in pairs]

    @functools.partial(
        pl.kernel,
        out_type=[jax.ShapeDtypeStruct((EE,), F32)] * P,
        mesh=_mesh(),
        compiler_params=_sc_params(),
        scratch_types=[pltpu.VMEM((NP,), F32),
                       pltpu.VMEM((WIN,), jnp.int32),
                       pltpu.VMEM((WIN,), F32)],
    )
    def k(*refs):
        tabs = refs[:P]
        idx_h = refs[P:2 * P]
        outs = refs[2 * P:3 * P]
        tab_v, idx_v, val_v = refs[3 * P:]
        wid = lax.axis_index("s") * NC + lax.axis_index("c")
        base = wid * EW
        for p in range(P):
            pltpu.sync_copy(tabs[p], tab_v)

            @pl.loop(0, EW, step=WIN)
            def _(off, p=p):
                pltpu.sync_copy(idx_h[p].at[pl.ds(base + off, WIN)], idx_v)

                @pl.loop(0, WIN - LN, step=LN)
                def _(j):
                    val_v[pl.ds(j, LN)] = plsc.load_gather(
                        tab_v, [idx_v[pl.ds(j, LN)]])

                val_v[pl.ds(WIN - LN, LN)] = plsc.load_gather(
                    tab_v, [idx_v[pl.ds(WIN - LN, LN)]])

                pltpu.sync_copy(val_v, outs[p].at[pl.ds(base + off, WIN)])

    res = k(*tables, *idxs)
    return res if P > 1 else [res]


def _sc_scatter_scalars(vals_list, idx, zeros_np):
    P = len(vals_list)

    @functools.partial(
        pl.kernel,
        out_type=[jax.ShapeDtypeStruct((NC * NP,), F32)] * P,
        mesh=_mesh(),
        compiler_params=_sc_params(),
        scratch_types=[pltpu.VMEM_SHARED((NP,), F32)] * P
        + [pltpu.VMEM((WIN,), jnp.int32), pltpu.VMEM((WIN,), F32),
           pltpu.VMEM((ROWS,), F32)],
    )
    def k(*refs):
        z_h = refs[0]
        vals_h = refs[1:1 + P]
        idx_h = refs[1 + P]
        outs = refs[2 + P:2 + 2 * P]
        accs = refs[2 + 2 * P:2 + 3 * P]
        idx_v = refs[2 + 3 * P]
        val_v = refs[3 + 3 * P]
        row_v = refs[4 + 3 * P]
        cid = lax.axis_index("c")
        sid = lax.axis_index("s")
        pltpu.sync_copy(z_h.at[pl.ds(sid * ROWS, ROWS)], row_v)
        for p in range(P):
            pltpu.sync_copy(row_v, accs[p].at[pl.ds(sid * ROWS, ROWS)])
        plsc.subcore_barrier()
        base = (sid * NC + cid) * EW

        @pl.loop(0, EW, step=WIN)
        def _(off):
            pltpu.sync_copy(idx_h.at[pl.ds(base + off, WIN)], idx_v)
            for p in range(P):
                pltpu.sync_copy(vals_h[p].at[pl.ds(base + off, WIN)], val_v)
                pltpu.sync_copy(val_v, accs[p].at[idx_v], add=True)

        plsc.subcore_barrier()
        for p in range(P):
            pltpu.sync_copy(accs[p].at[pl.ds(sid * ROWS, ROWS)], row_v)
            pltpu.sync_copy(row_v,
                            outs[p].at[pl.ds(cid * NP + sid * ROWS, ROWS)])

    res = k(zeros_np, *vals_list, idx)
    return res if P > 1 else [res]


def _sc_gather_rows(pairs):
    P = len(pairs)
    tables = [t for t, _ in pairs]
    idxs = [i for _, i in pairs]

    @functools.partial(
        pl.kernel,
        out_type=[jax.ShapeDtypeStruct((EE, 32), F32)] * P,
        mesh=_mesh(),
        compiler_params=_sc_params(),
        scratch_types=[pltpu.VMEM_SHARED((NP, 32), F32),
                       pltpu.VMEM((WINR,), jnp.int32),
                       pltpu.VMEM((WINR, 32), F32),
                       pltpu.VMEM((184, 32), F32)],
    )
    def k(*refs):
        tabs = refs[:P]
        idx_h = refs[P:2 * P]
        outs = refs[2 * P:3 * P]
        tab_sh, idx_v, row_v, stage_v = refs[3 * P:]
        cid = lax.axis_index("c")
        sid = lax.axis_index("s")
        base = (sid * NC + cid) * EW
        ch = 184
        for p in range(P):
            @pl.loop(0, ROWS, step=ch)
            def _(r, p=p):
                pltpu.sync_copy(tabs[p].at[pl.ds(sid * ROWS + r, ch)],
                                stage_v)
                pltpu.sync_copy(stage_v,
                                tab_sh.at[pl.ds(sid * ROWS + r, ch)])

            plsc.subcore_barrier()

            @pl.loop(0, EW, step=WINR)
            def _(off, p=p):
                pltpu.sync_copy(idx_h[p].at[pl.ds(base + off, WINR)], idx_v)
                pltpu.sync_copy(tab_sh.at[idx_v], row_v)
                pltpu.sync_copy(row_v, outs[p].at[pl.ds(base + off, WINR)])

            plsc.subcore_barrier()

    return k(*tables, *idxs)


def _sc_scatter_rows(wu0, wu1, idx, zeros_rows):
    @functools.partial(
        pl.kernel,
        out_type=[jax.ShapeDtypeStruct((NP, 32), F32)] * 2,
        mesh=_mesh(),
        compiler_params=_sc_params(),
        scratch_types=[pltpu.VMEM_SHARED((NP, 32), F32),
                       pltpu.VMEM((WINR,), jnp.int32),
                       pltpu.VMEM((WINR, 32), F32),
                       pltpu.VMEM((184, 32), F32)],
    )
    def k(z_h, wu0_h, wu1_h, idx_h, out0_h, out1_h, acc, idx_v, row_v,
          stage_v):
        cid = lax.axis_index("c")
        sid = lax.axis_index("s")
        ch = 184

        @pl.loop(0, ROWS, step=ch)
        def _(r):
            pltpu.sync_copy(z_h.at[pl.ds(sid * ROWS + r, ch)], stage_v)
            pltpu.sync_copy(stage_v, acc.at[pl.ds(sid * ROWS + r, ch)])

        plsc.subcore_barrier()
        ew = EE // NS
        base = sid * ew

        def run(wu_h, out_h):
            @pl.loop(0, ew, step=WINR)
            def _(off):
                pltpu.sync_copy(idx_h.at[pl.ds(base + off, WINR)], idx_v)
                pltpu.sync_copy(wu_h.at[pl.ds(base + off, WINR)], row_v)
                pltpu.sync_copy(row_v, acc.at[idx_v], add=True)

            plsc.subcore_barrier()

            @pl.loop(0, ROWS, step=ch)
            def _(r):
                pltpu.sync_copy(acc.at[pl.ds(sid * ROWS + r, ch)], stage_v)
                pltpu.sync_copy(stage_v, out_h.at[pl.ds(sid * ROWS + r, ch)])

        @pl.when(cid == 0)
        def _():
            run(wu0_h, out0_h)

        @pl.when(cid == 1)
        def _():
            run(wu1_h, out1_h)

    return k(zeros_rows, wu0, wu1, idx)


def _leaky(m):
    return jnp.where(m >= 0, m, 0.2 * m)


def _tc_elementwise(body, ins, out_shapes, specs_in, specs_out, grid):
    return pl.pallas_call(
        body,
        grid=grid,
        in_specs=specs_in,
        out_specs=specs_out,
        out_shape=[jax.ShapeDtypeStruct(s, F32) for s in out_shapes],
    )(*ins)


def _node_spec():
    return pl.BlockSpec((NPR, 128), lambda: (0, 0))


def _wspec():
    return pl.BlockSpec((1, 64), lambda: (0, 0))


def tc_node1(degP0, degP1, asumP0, asumP1, xp, wlr, we, bb, att):
    def body(d0, d1, a0, a1, x, wlr_r, we_r, bb_r, att_r,
             cnt_o, la_o, n0_o, n1_o):
        deg = d0[...] + d1[...]
        asum = a0[...] + a1[...]
        la = asum / jnp.maximum(deg, 1.0)
        cnt_o[...] = deg + 1.0
        la_o[...] = la
        x_ = x[...]
        acc0 = jnp.zeros_like(x_)
        acc1 = jnp.zeros_like(x_)
        for kk in range(64):
            m = x_ * wlr_r[0, kk] + la * we_r[0, kk] + bb_r[0, kk]
            m = _leaky(m) * att_r[0, kk]
            if kk < 32:
                acc0 = acc0 + m
            else:
                acc1 = acc1 + m
        n0_o[...] = acc0
        n1_o[...] = acc1

    return _tc_elementwise(
        body,
        [degP0, degP1, asumP0, asumP1, xp, wlr, we, bb, att],
        [(NPR, 128)] * 4,
        [_node_spec()] * 5 + [_wspec()] * 4,
        [_node_spec()] * 4,
        grid=(),
    )


_EB = 8192
_EBR = _EB // 128
_EGRID = (EE + _EB - 1) // _EB


def _edge_spec():
    return pl.BlockSpec((_EBR, 128), lambda i: (i, 0))


def tc_alpha1(xs, xd, ea, wl, wr, we, bb, att):
    def body(xs_r, xd_r, ea_r, wl_r, wr_r, we_r, bb_r, att_r, a0_o, a1_o):
        xs_ = xs_r[...]
        xd_ = xd_r[...]
        ea_ = ea_r[...]
        acc0 = jnp.zeros_like(xs_)
        acc1 = jnp.zeros_like(xs_)
        for kk in range(64):
            m = (xs_ * wl_r[0, kk] + xd_ * wr_r[0, kk]
                 + ea_ * we_r[0, kk] + bb_r[0, kk])
            m = _leaky(m) * att_r[0, kk]
            if kk < 32:
                acc0 = acc0 + m
            else:
                acc1 = acc1 + m
        a0_o[...] = acc0
        a1_o[...] = acc1

    return _tc_elementwise(
        body,
        [xs, xd, ea, wl, wr, we, bb, att],
        [(ER, 128)] * 2,
        [_edge_spec()] * 3 + [pl.BlockSpec((1, 64), lambda i: (0, 0))] * 5,
        [_edge_spec()] * 2,
        grid=(_EGRID,),
    )


def tc_mean1(salP0_0, salP1_0, salP0_1, salP1_1, n0, n1, cnt, xp):
    def body(s00, s10, s01, s11, n0_r, n1_r, cnt_r, x_r,
             m0_o, m1_o, id0_o, is0_o, id1_o, is1_o):
        cnt_ = cnt_r[...]
        x_ = x_r[...]
        for (sa, sb, n_r, m_o, id_o, is_o) in (
                (s00, s10, n0_r, m0_o, id0_o, is0_o),
                (s01, s11, n1_r, m1_o, id1_o, is1_o)):
            n_ = n_r[...]
            mean = (sa[...] + sb[...] + n_) / cnt_
            exn = jnp.exp(n_ - mean)
            m_o[...] = mean
            id_o[...] = exn
            is_o[...] = exn * x_

    return _tc_elementwise(
        body,
        [salP0_0, salP1_0, salP0_1, salP1_1, n0, n1, cnt, xp],
        [(NPR, 128)] * 6,
        [_node_spec()] * 8,
        [_node_spec()] * 6,
        grid=(),
    )


def tc_exp1(a0, a1, md0, md1, xs):
    def body(a0_r, a1_r, m0_r, m1_r, xs_r, e0_o, es0_o, e1_o, es1_o):
        xs_ = xs_r[...]
        e0 = jnp.exp(a0_r[...] - m0_r[...])
        e1 = jnp.exp(a1_r[...] - m1_r[...])
        e0_o[...] = e0
        es0_o[...] = e0 * xs_
        e1_o[...] = e1
        es1_o[...] = e1 * xs_

    return _tc_elementwise(
        body,
        [a0, a1, md0, md1, xs],
        [(ER, 128)] * 4,
        [_edge_spec()] * 5,
        [_edge_spec()] * 4,
        grid=(_EGRID,),
    )


_BN = 6144
_BNR = _BN // 128
_NGRID = (NP + _BN - 1) // _BN


def _colspec():
    return pl.BlockSpec((1, _BN), lambda i: (0, i))


def _rowspec32():
    return pl.BlockSpec((_BN, 32), lambda i: (i, 0))


def _col(v):
    return jnp.transpose(v[...])


def tc_h1(denP00, denP10, SP00, SP10, denP01, denP11, SP01, SP11,
          iden0, iS0, iden1, iS1, la,
          Wl1, bl1, bias1, Wl2, bl2, Wr2, br2, We2, att2):
    def body(d00, d10, s00, s10, d01, d11, s01, s11,
             i0, j0, i1, j1, la_r,
             wl1_r, bl1_r, b1_r, wl2_r, bl2_r, wr2_r, br2_r, we2_r, att2_r,
             u0_o, u1_o, v0_o, v1_o, a2n0_o, a2n1_o):
        den0 = _col(d00) + _col(d10) + _col(i0)
        S0 = _col(s00) + _col(s10) + _col(j0)
        den1 = _col(d01) + _col(d11) + _col(i1)
        S1 = _col(s01) + _col(s11) + _col(j1)
        A0 = S0 / (den0 + 1e-16)
        SA0 = den0 / (den0 + 1e-16)
        A1 = S1 / (den1 + 1e-16)
        SA1 = den1 / (den1 + 1e-16)
        wl1 = wl1_r[...]
        bl1_ = bl1_r[...]
        h1a = A0 * wl1[:, :32] + SA0 * bl1_[:, :32] + b1_r[:, :32]
        h1b = A1 * wl1[:, 32:] + SA1 * bl1_[:, 32:] + b1_r[:, 32:]
        h1 = jnp.maximum(jnp.concatenate([h1a, h1b], axis=1), 0.0)
        xl2 = jax.lax.dot_general(
            h1, wl2_r[...], (((1,), (0,)), ((), ())),
            preferred_element_type=F32) + bl2_r[...]
        xr2 = jax.lax.dot_general(
            h1, wr2_r[...], (((1,), (0,)), ((), ())),
            preferred_element_type=F32) + br2_r[...]
        u0_o[...] = xl2[:, :32]
        u1_o[...] = xl2[:, 32:]
        v0_o[...] = xr2[:, :32]
        v1_o[...] = xr2[:, 32:]
        m2n = _leaky(xl2 + xr2 + _col(la_r) * we2_r[...]) * att2_r[...]
        a2n0 = jnp.sum(m2n[:, :32], axis=1, keepdims=True)
        a2n1 = jnp.sum(m2n[:, 32:], axis=1, keepdims=True)
        a2n0_o[...] = jnp.transpose(a2n0)
        a2n1_o[...] = jnp.transpose(a2n1)

    wspec = pl.BlockSpec((1, 64), lambda i: (0, 0))
    mspec = pl.BlockSpec((64, 64), lambda i: (0, 0))
    return pl.pallas_call(
        body,
        grid=(_NGRID,),
        in_specs=[_colspec()] * 13 + [wspec, wspec, wspec, mspec, wspec,
                                      mspec, wspec, wspec, wspec],
        out_specs=[_rowspec32()] * 4 + [_colspec()] * 2,
        out_shape=[jax.ShapeDtypeStruct((NP, 32), F32)] * 4
        + [jax.ShapeDtypeStruct((1, NP), F32)] * 2,
    )(denP00, denP10, SP00, SP10, denP01, denP11, SP01, SP11,
      iden0, iS0, iden1, iS1, la,
      Wl1, bl1, bias1, Wl2, bl2, Wr2, br2, We2, att2)


def _erowspec():
    return pl.BlockSpec((_EB, 32), lambda i: (i, 0))


def _escalarspec():
    return pl.BlockSpec((1, _EB), lambda i: (0, i))


def tc_alpha2(u0, v0, u1, v1, ea, We2, att2):
    def body(u0_r, v0_r, u1_r, v1_r, ea_r, we_r, att_r, a0_o, a1_o):
        eac = jnp.transpose(ea_r[...])
        we = we_r[...]
        att = att_r[...]
        m0 = _leaky(u0_r[...] + v0_r[...] + eac * we[:, :32]) * att[:, :32]
        m1 = _leaky(u1_r[...] + v1_r[...] + eac * we[:, 32:]) * att[:, 32:]
        a0 = jnp.sum(m0, axis=1, keepdims=True)
        a1 = jnp.sum(m1, axis=1, keepdims=True)
        a0_o[...] = jnp.transpose(a0)
        a1_o[...] = jnp.transpose(a1)

    wspec = pl.BlockSpec((1, 64), lambda i: (0, 0))
    return pl.pallas_call(
        body,
        grid=(_EGRID,),
        in_specs=[_erowspec()] * 4 + [_escalarspec(), wspec, wspec],
        out_specs=[_escalarspec()] * 2,
        out_shape=[jax.ShapeDtypeStruct((1, EE), F32)] * 2,
    )(u0, v0, u1, v1, ea, We2, att2)


def tc_mean2(salP0_0, salP1_0, salP0_1, salP1_1, n0, n1, cnt):
    def body(s00, s10, s01, s11, n0_r, n1_r, cnt_r,
             m0_o, m1_o, x0_o, x1_o):
        cnt_ = cnt_r[...]
        for (sa, sb, n_r, m_o, x_o) in ((s00, s10, n0_r, m0_o, x0_o),
                                        (s01, s11, n1_r, m1_o, x1_o)):
            n_ = n_r[...]
            mean = (sa[...] + sb[...] + n_) / cnt_
            m_o[...] = mean
            x_o[...] = jnp.exp(n_ - mean)

    return _tc_elementwise(
        body,
        [salP0_0, salP1_0, salP0_1, salP1_1, n0, n1, cnt],
        [(NPR, 128)] * 4,
        [_node_spec()] * 7,
        [_node_spec()] * 4,
        grid=(),
    )


def tc_w2(a0, a1, md0, md1, u0, u1):
    def body(a0_r, a1_r, m0_r, m1_r, u0_r, u1_r,
             e0_o, e1_o, w0_o, w1_o):
        e0 = jnp.exp(a0_r[...] - m0_r[...])
        e1 = jnp.exp(a1_r[...] - m1_r[...])
        e0_o[...] = e0
        e1_o[...] = e1
        w0_o[...] = u0_r[...] * jnp.transpose(e0)
        w1_o[...] = u1_r[...] * jnp.transpose(e1)

    return pl.pallas_call(
        body,
        grid=(_EGRID,),
        in_specs=[_escalarspec()] * 4 + [_erowspec()] * 2,
        out_specs=[_escalarspec()] * 2 + [_erowspec()] * 2,
        out_shape=[jax.ShapeDtypeStruct((1, EE), F32)] * 2
        + [jax.ShapeDtypeStruct((EE, 32), F32)] * 2,
    )(a0, a1, md0, md1, u0, u1)


def tc_readout(r0, r1, d2P00, d2P10, d2P01, d2P11, xn0, xn1, u0, u1,
               bias2, message, Wfc, bfc):
    def body(r0_r, r1_r, d00, d10, d01, d11, x0, x1, u0_r, u1_r,
             b2_r, msg_r, wfc_r, bfc_r, out_o):
        xn0_ = _col(x0)
        xn1_ = _col(x1)
        den0 = _col(d00) + _col(d10) + xn0_ + 1e-16
        den1 = _col(d01) + _col(d11) + xn1_ + 1e-16
        o0 = (r0_r[...] + xn0_ * u0_r[...]) / den0
        o1 = (r1_r[...] + xn1_ * u1_r[...]) / den1
        h2 = jnp.maximum(
            jnp.concatenate([o0, o1], axis=1) + b2_r[...], 0.0)
        me = jax.lax.dot_general(
            msg_r[...], wfc_r[...], (((1,), (1,)), ((), ())),
            preferred_element_type=F32) + bfc_r[...]
        dots = jax.lax.dot_general(
            h2, me, (((1,), (1,)), ((), ())), preferred_element_type=F32)
        mx = jnp.max(dots, axis=1, keepdims=True)
        p = jnp.exp(dots - mx)
        out_o[...] = p / jnp.sum(p, axis=1, keepdims=True)

    wspec = pl.BlockSpec((1, 64), lambda i: (0, 0))
    return pl.pallas_call(
        body,
        grid=(_NGRID,),
        in_specs=[_rowspec32()] * 2 + [_colspec()] * 6 + [_rowspec32()] * 2
        + [wspec,
           pl.BlockSpec((256, 128), lambda i: (0, 0)),
           pl.BlockSpec((64, 128), lambda i: (0, 0)),
           wspec],
        out_specs=pl.BlockSpec((_BN, 256), lambda i: (i, 0)),
        out_shape=jax.ShapeDtypeStruct((NN, 256), F32),
    )(r0, r1, d2P00, d2P10, d2P01, d2P11, xn0, xn1, u0, u1,
      bias2, message, Wfc, bfc)


def kernel(message, x, edge_index, edge_attr, Wl1, bl1, Wr1, br1, We1, att1,
           bias1, Wl2, bl2, Wr2, br2, We2, att2, bias2, Wfc, bfc):
    xp = jnp.pad(x[:, 0], (0, NP - NN))
    src = edge_index[0]
    dst = edge_index[1]
    ea = edge_attr[:, 0]
    zeros_np = jnp.zeros((NP,), F32)
    zeros_rows = jnp.zeros((NP, 32), F32)
    ones_e = jnp.ones((EE,), F32)

    nv = lambda a: a.reshape(NPR, 128)
    rv = lambda a: a.reshape(1, NP)
    ev = lambda a: a.reshape(ER, 128)
    er = lambda a: a.reshape(1, EE)
    w = lambda a: a.reshape(1, 64)

    wl1 = w(Wl1)
    wr1 = w(Wr1)
    we1 = w(We1)
    bb1 = w(bl1 + br1)
    at1 = w(att1.reshape(-1))
    we2 = w(We2)
    at2 = w(att2.reshape(-1))

    degP, asumP = _sc_scatter_scalars([ones_e, ea], dst, zeros_np)
    xs, xd = _sc_gather_scalars([(xp, src), (xp, dst)])
    cnt, la, n1_0, n1_1 = tc_node1(
        nv(degP[:NP]), nv(degP[NP:]), nv(asumP[:NP]), nv(asumP[NP:]),
        nv(xp), w(Wl1 + Wr1), we1, bb1, at1)
    a1_0, a1_1 = tc_alpha1(ev(xs), ev(xd), ev(ea), wl1, wr1, we1, bb1, at1)
    salP_0, salP_1 = _sc_scatter_scalars(
        [a1_0.reshape(-1), a1_1.reshape(-1)], dst, zeros_np)
    m1_0, m1_1, iden0, iS0, iden1, iS1 = tc_mean1(
        nv(salP_0[:NP]), nv(salP_0[NP:]), nv(salP_1[:NP]), nv(salP_1[NP:]),
        n1_0, n1_1, cnt, nv(xp))
    md0, md1 = _sc_gather_scalars(
        [(m1_0.reshape(-1), dst), (m1_1.reshape(-1), dst)])
    ex0, exs0, ex1, exs1 = tc_exp1(a1_0, a1_1, ev(md0), ev(md1), ev(xs))
    denP_0, SP_0, denP_1, SP_1 = _sc_scatter_scalars(
        [ex0.reshape(-1), exs0.reshape(-1), ex1.reshape(-1),
         exs1.reshape(-1)], dst, zeros_np)

    u0t, u1t, v0t, v1t, a2n0, a2n1 = tc_h1(
        rv(denP_0[:NP]), rv(denP_0[NP:]), rv(SP_0[:NP]), rv(SP_0[NP:]),
        rv(denP_1[:NP]), rv(denP_1[NP:]), rv(SP_1[:NP]), rv(SP_1[NP:]),
        rv(iden0), rv(iS0), rv(iden1), rv(iS1), rv(la),
        wl1, w(bl1), w(bias1), Wl2, w(bl2), Wr2, w(br2), we2, at2)

    u0g, u1g, v0g, v1g = _sc_gather_rows(
        [(u0t, src), (u1t, src), (v0t, dst), (v1t, dst)])
    a2_0, a2_1 = tc_alpha2(u0g, v0g, u1g, v1g, er(ea), we2, at2)
    sal2_0, sal2_1 = _sc_scatter_scalars(
        [a2_0.reshape(-1), a2_1.reshape(-1)], dst, zeros_np)
    m2_0, m2_1, xn2_0, xn2_1 = tc_mean2(
        nv(sal2_0[:NP]), nv(sal2_0[NP:]), nv(sal2_1[:NP]), nv(sal2_1[NP:]),
        nv(a2n0), nv(a2n1), cnt)
    md2_0, md2_1 = _sc_gather_scalars(
        [(m2_0.reshape(-1), dst), (m2_1.reshape(-1), dst)])
    ex2_0, ex2_1, wu0, wu1 = tc_w2(a2_0, a2_1, er(md2_0), er(md2_1),
                                   u0g, u1g)
    d2P_0, d2P_1 = _sc_scatter_scalars(
        [ex2_0.reshape(-1), ex2_1.reshape(-1)], dst, zeros_np)
    r0, r1 = _sc_scatter_rows(wu0, wu1, dst, zeros_rows)

    return tc_readout(
        r0, r1, rv(d2P_0[:NP]), rv(d2P_0[NP:]), rv(d2P_1[:NP]),
        rv(d2P_1[NP:]), rv(xn2_0), rv(xn2_1), u0t, u1t,
        w(bias2), message, Wfc, w(bfc))
